# baseline (speedup 1.0000x reference)
import functools

import jax
import jax.numpy as jnp
from jax import lax
from jax.experimental import pallas as pl
from jax.experimental.pallas import tpu as pltpu
from jax.experimental.pallas import tpu_sc as plsc

B = 16
NPG = 1024
N = B * NPG
E = N * 16
C = 100
F32 = jnp.float32
EPG = E // B
RB = NPG // 32
L = 16


EH = EPG // 2


def _rnd_bf16_bits(v):
    u = plsc.bitcast(v, jnp.int32)
    lsb = lax.shift_right_logical(u, 16) & 1
    return lax.shift_right_logical(u + 0x7FFF + lsb, 16)


def _sc_build_kernel(src_hbm, dst_hbm, w_hbm, mw_hbm, m1_hbm,
                     srcb, dstb, wb, mwb, m1b, pkb,
                     sem0, sem1, sem2):
    wid = lax.axis_index("c") * 16 + lax.axis_index("s")
    row0 = wid * RB
    ones = jnp.full((L,), 1.0, F32)

    def per_graph(g, _):
        goff = g * NPG + row0

        def scan_half(h, _):
            base = g * EPG + h * EH
            c0 = pltpu.async_copy(src_hbm.at[pl.ds(base, EH)], srcb, sem0)
            c1 = pltpu.async_copy(dst_hbm.at[pl.ds(base, EH)], dstb, sem1)
            c2 = pltpu.async_copy(w_hbm.at[pl.ds(base, EH)], wb, sem2)

            @pl.when(h == 0)
            def _zero():
                def zero_chunk(i, _):
                    z = jnp.zeros((L,), F32)
                    mwb[pl.ds(i * L, L)] = z
                    m1b[pl.ds(i * L, L)] = z
                    return 0
                lax.fori_loop(0, (RB * NPG) // L, zero_chunk, 0, unroll=8)

            c0.wait()
            c1.wait()
            c2.wait()

            def scan_chunk(e, _):
                sv = srcb[pl.ds(e * L, L)]
                dv = dstb[pl.ds(e * L, L)]
                wv = wb[pl.ds(e * L, L)]
                r = dv - goff
                mask = (r >= 0) & (r < RB)
                sl = sv - g * NPG
                slp = (sl & ~31) | ((sl & 31) >> 1) | ((sl & 1) << 4)
                flat = jnp.where(mask, r * NPG + slp, 0)
                plsc.addupdate_scatter(mwb, [flat], wv, mask=mask)
                plsc.addupdate_scatter(m1b, [flat], ones, mask=mask)
                return 0

            lax.fori_loop(0, EH // L, scan_chunk, 0, unroll=4)
            return 0

        lax.fori_loop(0, 2, scan_half, 0)

        hoff = g * (NPG * NPG // 2) + wid * (RB * NPG // 2)

        def flush(blk, out_hbm):
            def pack_chunk(i, _):
                a = blk[pl.ds(i * 2 * L, L)]
                b = blk[pl.ds(i * 2 * L + L, L)]
                pkb[pl.ds(i * L, L)] = (_rnd_bf16_bits(a)
                                        | lax.shift_left(_rnd_bf16_bits(b), 16))
                return 0
            lax.fori_loop(0, (RB * NPG) // (2 * L), pack_chunk, 0, unroll=8)
            pltpu.sync_copy(pkb, out_hbm.at[pl.ds(hoff, RB * NPG // 2)])

        flush(mwb, mw_hbm)
        flush(m1b, m1_hbm)
        return 0

    lax.fori_loop(0, B, per_graph, 0)


def _sc_build(src, dst, w):
    mesh = plsc.VectorSubcoreMesh(core_axis_name="c", subcore_axis_name="s")
    f = pl.kernel(
        _sc_build_kernel,
        mesh=mesh,
        compiler_params=pltpu.CompilerParams(needs_layout_passes=False),
        out_type=[jax.ShapeDtypeStruct((B * NPG * NPG // 2,), jnp.int32),
                  jax.ShapeDtypeStruct((B * NPG * NPG // 2,), jnp.int32)],
        scratch_types=[pltpu.VMEM((EH,), jnp.int32),
                       pltpu.VMEM((EH,), jnp.int32),
                       pltpu.VMEM((EH,), F32),
                       pltpu.VMEM((RB * NPG,), F32),
                       pltpu.VMEM((RB * NPG,), F32),
                       pltpu.VMEM((RB * NPG // 2,), jnp.int32),
                       pltpu.SemaphoreType.DMA,
                       pltpu.SemaphoreType.DMA,
                       pltpu.SemaphoreType.DMA],
    )
    mw, m1 = f(src, dst, w)
    tobf = lambda a: jax.lax.bitcast_convert_type(
        a, jnp.bfloat16).reshape(B, NPG, NPG)
    return tobf(mw), tobf(m1)


def _dot(a, b):
    return jax.lax.dot_general(a, b, (((1,), (0,)), ((), ())),
                               preferred_element_type=F32)


def _dot_t(a, b):
    return jax.lax.dot_general(a, b, (((0,), (0,)), ((), ())),
                               preferred_element_type=F32)


def _fold(sum_ref, sq_ref, g_ref, be_ref, n):
    m = jnp.sum(sum_ref[...], axis=0) / n
    var = jnp.sum(sq_ref[...], axis=0) / n - m * m
    a = g_ref[...] / jnp.sqrt(var + 1e-5)
    c = be_ref[...] - m * a
    return a, c


def _branch(m, dis, h, w_ref, b_ref):
    v = _dot(h, w_ref[...])
    vw = dis[:, None] * v
    u = dis[:, None] * (_dot(m, vw) + vw) + b_ref[...]
    return u


def _write_stats(u, u_ref, sum_ref, sq_ref, mx_ref, mn_ref):
    f = u.shape[1]
    u_ref[...] = u
    sum_ref[...] = jnp.sum(u, axis=0).reshape(1, 1, f)
    sq_ref[...] = jnp.sum(u * u, axis=0).reshape(1, 1, f)
    mx_ref[...] = jnp.max(u, axis=0).reshape(1, 1, f)
    mn_ref[...] = jnp.min(u, axis=0).reshape(1, 1, f)


def _stage1_kernel(mw_ref, m1_ref, x_ref, wx_ref, bx_ref, ws_ref, bs_ref,
                   ux_ref, sx_ref, qx_ref, mxx_ref, mnx_ref,
                   us_ref, ss_ref, qs_ref, mxs_ref, mns_ref):
    mw = mw_ref[0].astype(F32)
    m1 = m1_ref[0].astype(F32)
    disw = lax.rsqrt(jnp.sum(mw, axis=1) + 1.0)
    dis1 = lax.rsqrt(jnp.sum(m1, axis=1) + 1.0)
    h = x_ref[...]
    ux = _branch(mw, disw, h, wx_ref, bx_ref)
    _write_stats(ux, ux_ref, sx_ref, qx_ref, mxx_ref, mnx_ref)
    us = _branch(m1, dis1, h, ws_ref, bs_ref)
    _write_stats(us, us_ref, ss_ref, qs_ref, mxs_ref, mns_ref)


def _stage_kernel(mw_ref, m1_ref, hx_ref, sxp_ref, qxp_ref, gxp_ref, bexp_ref,
                  hs_ref, ssp_ref, qsp_ref, gsp_ref, besp_ref,
                  wx_ref, bx_ref, ws_ref, bs_ref,
                  ux_ref, sx_ref, qx_ref, mxx_ref, mnx_ref,
                  us_ref, ss_ref, qs_ref, mxs_ref, mns_ref):
    mw = mw_ref[0].astype(F32)
    m1 = m1_ref[0].astype(F32)
    disw = lax.rsqrt(jnp.sum(mw, axis=1) + 1.0)
    dis1 = lax.rsqrt(jnp.sum(m1, axis=1) + 1.0)
    ax, cx = _fold(sxp_ref, qxp_ref, gxp_ref, bexp_ref, float(N))
    hx = hx_ref[...] * ax + cx
    ux = _branch(mw, disw, hx, wx_ref, bx_ref)
    _write_stats(ux, ux_ref, sx_ref, qx_ref, mxx_ref, mnx_ref)
    as_, cs = _fold(ssp_ref, qsp_ref, gsp_ref, besp_ref, float(N))
    hs = hs_ref[...] * as_ + cs
    us = _branch(m1, dis1, hs, ws_ref, bs_ref)
    _write_stats(us, us_ref, ss_ref, qs_ref, mxs_ref, mns_ref)


def _pool_kernel(mw_ref, m1_ref, ux3_ref, sx3_ref, qx3_ref, g13_ref, be13_ref,
                 us1_ref, ss1_ref, qs1_ref, gp1_ref, bep1_ref,
                 us2_ref, ss2_ref, qs2_ref, gp2_ref, bep2_ref,
                 us3_ref, ss3_ref, qs3_ref, gp3_ref, bep3_ref,
                 wpf1_ref, wpf2_ref, wpf3_ref, bpf_ref,
                 padj_ref, px_ref, misc_ref):
    a1, c1 = _fold(ss1_ref, qs1_ref, gp1_ref, bep1_ref, float(N))
    a2, c2 = _fold(ss2_ref, qs2_ref, gp2_ref, bep2_ref, float(N))
    a3, c3 = _fold(ss3_ref, qs3_ref, gp3_ref, bep3_ref, float(N))
    s1 = (_dot(us1_ref[...] * a1 + c1, wpf1_ref[...])
          + _dot(us2_ref[...] * a2 + c2, wpf2_ref[...])
          + _dot(us3_ref[...] * a3 + c3, wpf3_ref[...])
          + bpf_ref[...])
    mx = jnp.max(s1, axis=1, keepdims=True)
    ex = jnp.exp(s1 - mx)
    ss = ex / jnp.sum(ex, axis=1, keepdims=True)
    el = -jnp.sum(ss * jnp.log(ss + 1e-15))
    t1 = _dot(m1_ref[0].astype(F32), ss)
    ml = jnp.sum(ss * t1)
    tw = _dot(mw_ref[0].astype(F32), ss)
    padj_ref[0] = _dot_t(tw, ss)
    ax3, cx3 = _fold(sx3_ref, qx3_ref, g13_ref, be13_ref, float(N))
    x13bn = ux3_ref[...] * ax3 + cx3
    px_ref[0] = _dot_t(ss, x13bn)
    misc_ref[...] = jnp.concatenate(
        [el.reshape(1, 1), ml.reshape(1, 1)], axis=1).reshape(1, 1, 2)


def _maxmin_chunk(mx, mn, a, c):
    return jnp.where(a > 0, a * mx, a * mn) + c


def _head_kernel(padj_ref, px_ref, misc_ref,
                 mxx1_ref, mnx1_ref, sx1_ref, qx1_ref, g11_ref, be11_ref,
                 mxx2_ref, mnx2_ref, sx2_ref, qx2_ref, g12_ref, be12_ref,
                 mxx3_ref, mnx3_ref, sx3_ref, qx3_ref, g13_ref, be13_ref,
                 w21_ref, b21_ref, g21_ref, be21_ref,
                 w22_ref, b22_ref, g22_ref, be22_ref,
                 w23_ref, b23_ref, g23_ref, be23_ref,
                 wf1_ref, bf1_ref, wf2_ref, bf2_ref,
                 out_ref, reg_ref):
    n2 = float(B * C)
    chunks = []
    for mxr, mnr, sr, qr, gr, ber in (
            (mxx1_ref, mnx1_ref, sx1_ref, qx1_ref, g11_ref, be11_ref),
            (mxx2_ref, mnx2_ref, sx2_ref, qx2_ref, g12_ref, be12_ref),
            (mxx3_ref, mnx3_ref, sx3_ref, qx3_ref, g13_ref, be13_ref)):
        a, c = _fold(sr, qr, gr, ber, float(N))
        chunks.append(_maxmin_chunk(mxr[...].reshape(B, -1),
                                    mnr[...].reshape(B, -1), a, c))
    x1_out = jnp.concatenate(chunks, axis=1)

    dis2 = []
    for g in range(B):
        deg = jnp.sum(padj_ref[g], axis=0, keepdims=True) + 1.0
        dis2.append(jnp.where(deg > 0, lax.rsqrt(deg), 0.0))

    def layer2(hs, w_ref, b_ref):
        us = []
        for g in range(B):
            v = _dot(hs[g], w_ref[...])
            vw = dis2[g].reshape(C, 1) * v
            u = dis2[g].reshape(C, 1) * (_dot_t(padj_ref[g], vw) + vw) \
                + b_ref[...]
            us.append(u)
        flat = jnp.concatenate(us, axis=0)
        s = jnp.sum(flat, axis=0, keepdims=True) / n2
        var = jnp.sum(flat * flat, axis=0, keepdims=True) / n2 - s * s
        return us, s, var

    hs = [px_ref[g] for g in range(B)]
    x2_chunks = []
    for w_ref, b_ref, g_ref, be_ref in (
            (w21_ref, b21_ref, g21_ref, be21_ref),
            (w22_ref, b22_ref, g22_ref, be22_ref),
            (w23_ref, b23_ref, g23_ref, be23_ref)):
        us, m, var = layer2(hs, w_ref, b_ref)
        a = g_ref[...] / jnp.sqrt(var + 1e-5)
        c = be_ref[...] - m * a
        mxs = jnp.concatenate(
            [jnp.max(u, axis=0, keepdims=True) for u in us], axis=0)
        mns = jnp.concatenate(
            [jnp.min(u, axis=0, keepdims=True) for u in us], axis=0)
        x2_chunks.append(_maxmin_chunk(mxs, mns, a, c))
        hs = [u * a + c for u in us]
    x2_out = jnp.concatenate(x2_chunks, axis=1)

    conv = jnp.concatenate([x1_out, x2_out], axis=1)
    h = jnp.maximum(_dot(conv, wf1_ref[...]) + bf1_ref[...], 0.0)
    out_ref[...] = _dot(h, wf2_ref[...]) + bf2_ref[...]
    misc = misc_ref[...].reshape(B, 2)
    reg = (jnp.sum(misc[:, 0]) / float(N)) - (jnp.sum(misc[:, 1]) / float(E))
    reg_ref[...] = reg.reshape(1, 1)


def _full(shape):
    nd = len(shape)
    return pl.BlockSpec(shape, lambda g, _nd=nd: (0,) * _nd)


def _gblk(shape):
    nd = len(shape)
    return pl.BlockSpec((1,) + shape[1:],
                        lambda g, _nd=nd: (g,) + (0,) * (_nd - 1))


def _nblk(f):
    return pl.BlockSpec((NPG, f), lambda g: (g, 0))


def _stage_out(fx, fs):
    shapes = [jax.ShapeDtypeStruct((N, fx), F32)] + \
             [jax.ShapeDtypeStruct((B, 1, fx), F32)] * 4 + \
             [jax.ShapeDtypeStruct((N, fs), F32)] + \
             [jax.ShapeDtypeStruct((B, 1, fs), F32)] * 4
    specs = [_nblk(fx)] + [_gblk((B, 1, fx))] * 4 + \
            [_nblk(fs)] + [_gblk((B, 1, fs))] * 4
    return shapes, specs


def kernel(x, edge_index, edge_attr, params):
    p = params
    mw, m1 = _sc_build(edge_index[0], edge_index[1], edge_attr)

    def row(name):
        return p[name].reshape(1, -1)

    adj_spec = pl.BlockSpec((1, NPG, NPG), lambda g: (g, 0, 0))

    shapes, ospecs = _stage_out(30, 30)
    s1out = pl.pallas_call(
        _stage1_kernel,
        grid=(B,),
        in_specs=[adj_spec, adj_spec, _nblk(3),
                  _full((3, 30)), _full((1, 30)),
                  _full((3, 30)), _full((1, 30))],
        out_specs=ospecs,
        out_shape=shapes,
    )(mw, m1, x, p['W11'], row('b11'), p['Wp11'], row('bp11'))
    (ux1, sx1, qx1, mxx1, mnx1, us1, ss1, qs1, mxs1, mns1) = s1out

    def stage(fx_in, fs_in, fx_out, fs_out, hx, sxp, qxp, gxp, bexp,
              hs, ssp, qsp, gsp, besp, wx, bx, ws, bs):
        shapes, ospecs = _stage_out(fx_out, fs_out)
        return pl.pallas_call(
            _stage_kernel,
            grid=(B,),
            in_specs=[adj_spec, adj_spec,
                      _nblk(fx_in), _full((B, 1, fx_in)), _full((B, 1, fx_in)),
                      _full((1, fx_in)), _full((1, fx_in)),
                      _nblk(fs_in), _full((B, 1, fs_in)), _full((B, 1, fs_in)),
                      _full((1, fs_in)), _full((1, fs_in)),
                      _full((fx_in, fx_out)), _full((1, fx_out)),
                      _full((fs_in, fs_out)), _full((1, fs_out))],
            out_specs=ospecs,
            out_shape=shapes,
        )(mw, m1, hx, sxp, qxp, gxp, bexp, hs, ssp, qsp, gsp, besp,
          wx, bx, ws, bs)

    (ux2, sx2, qx2, mxx2, mnx2, us2, ss2, qs2, _, _) = stage(
        30, 30, 30, 30, ux1, sx1, qx1, row('g11'), row('be11'),
        us1, ss1, qs1, row('gp11'), row('bep11'),
        p['W12'], row('b12'), p['Wp12'], row('bp12'))
    (ux3, sx3, qx3, mxx3, mnx3, us3, ss3, qs3, _, _) = stage(
        30, 30, 30, 100, ux2, sx2, qx2, row('g12'), row('be12'),
        us2, ss2, qs2, row('gp12'), row('bep12'),
        p['W13'], row('b13'), p['Wp13'], row('bp13'))

    wpf = p['Wpf']
    padj, px, misc = pl.pallas_call(
        _pool_kernel,
        grid=(B,),
        in_specs=[adj_spec, adj_spec,
                  _nblk(30), _full((B, 1, 30)), _full((B, 1, 30)),
                  _full((1, 30)), _full((1, 30)),
                  _nblk(30), _full((B, 1, 30)), _full((B, 1, 30)),
                  _full((1, 30)), _full((1, 30)),
                  _nblk(30), _full((B, 1, 30)), _full((B, 1, 30)),
                  _full((1, 30)), _full((1, 30)),
                  _nblk(100), _full((B, 1, 100)), _full((B, 1, 100)),
                  _full((1, 100)), _full((1, 100)),
                  _full((30, 100)), _full((30, 100)), _full((100, 100)),
                  _full((1, 100))],
        out_specs=[_gblk((B, C, C)), _gblk((B, C, 30)), _gblk((B, 1, 2))],
        out_shape=[jax.ShapeDtypeStruct((B, C, C), F32),
                   jax.ShapeDtypeStruct((B, C, 30), F32),
                   jax.ShapeDtypeStruct((B, 1, 2), F32)],
    )(mw, m1, ux3, sx3, qx3, row('g13'), row('be13'),
      us1, ss1, qs1, row('gp11'), row('bep11'),
      us2, ss2, qs2, row('gp12'), row('bep12'),
      us3, ss3, qs3, row('gp13'), row('bep13'),
      wpf[0:30], wpf[30:60], wpf[60:160], row('bpf'))

    args = [padj, px, misc,
            mxx1, mnx1, sx1, qx1, row('g11'), row('be11'),
            mxx2, mnx2, sx2, qx2, row('g12'), row('be12'),
            mxx3, mnx3, sx3, qx3, row('g13'), row('be13'),
            p['W21'], row('b21'), row('g21'), row('be21'),
            p['W22'], row('b22'), row('g22'), row('be22'),
            p['W23'], row('b23'), row('g23'), row('be23'),
            p['Wf1'], row('bf1'), p['Wf2'], row('bf2')]
    out, reg = pl.pallas_call(
        _head_kernel,
        out_shape=[jax.ShapeDtypeStruct((B, 6), F32),
                   jax.ShapeDtypeStruct((1, 1), F32)],
    )(*args)
    return (out, reg.reshape(()))


# speedup vs baseline: 1.6703x; 1.6703x over previous
import functools

import jax
import jax.numpy as jnp
from jax import lax
from jax.experimental import pallas as pl
from jax.experimental.pallas import tpu as pltpu
from jax.experimental.pallas import tpu_sc as plsc

B = 16
NPG = 1024
N = B * NPG
E = N * 16
C = 100
F32 = jnp.float32
EPG = E // B
RB = NPG // 32
L = 16


EH = EPG // 2


def _sc_build_kernel(src_hbm, dst_hbm, w_hbm, mw_hbm, m1_hbm,
                     srcb, dstb, wb, mwb, m1b,
                     sem0, sem1, sem2):
    wid = lax.axis_index("c") * 16 + lax.axis_index("s")
    row0 = wid * RB
    ones = jnp.full((L,), 1.0, F32)

    def per_graph(g, _):
        goff = g * NPG + row0

        def scan_half(h, _):
            base = g * EPG + h * EH
            c0 = pltpu.async_copy(src_hbm.at[pl.ds(base, EH)], srcb, sem0)
            c1 = pltpu.async_copy(dst_hbm.at[pl.ds(base, EH)], dstb, sem1)
            c2 = pltpu.async_copy(w_hbm.at[pl.ds(base, EH)], wb, sem2)

            @pl.when(h == 0)
            def _zero():
                def zero_chunk(i, _):
                    z = jnp.zeros((L,), F32)
                    mwb[pl.ds(i * L, L)] = z
                    m1b[pl.ds(i * L, L)] = z
                    return 0
                lax.fori_loop(0, (RB * NPG) // L, zero_chunk, 0, unroll=8)

            c0.wait()
            c1.wait()
            c2.wait()

            def scan_chunk(e, _):
                sv = srcb[pl.ds(e * L, L)]
                dv = dstb[pl.ds(e * L, L)]
                wv = wb[pl.ds(e * L, L)]
                r = dv - goff
                mask = (r >= 0) & (r < RB)
                sl = sv - g * NPG
                flat = jnp.where(mask, r * NPG + sl, 0)
                plsc.addupdate_scatter(mwb, [flat], wv, mask=mask)
                plsc.addupdate_scatter(m1b, [flat], ones, mask=mask)
                return 0

            lax.fori_loop(0, EH // L, scan_chunk, 0, unroll=4)
            return 0

        lax.fori_loop(0, 2, scan_half, 0)

        hoff_w = g * NPG * NPG + wid * (RB * NPG)
        pltpu.sync_copy(mwb, mw_hbm.at[pl.ds(hoff_w, RB * NPG)])
        pltpu.sync_copy(m1b, m1_hbm.at[pl.ds(hoff_w, RB * NPG)])
        return 0

    lax.fori_loop(0, B, per_graph, 0)


def _sc_build(src, dst, w):
    mesh = plsc.VectorSubcoreMesh(core_axis_name="c", subcore_axis_name="s")
    f = pl.kernel(
        _sc_build_kernel,
        mesh=mesh,
        compiler_params=pltpu.CompilerParams(needs_layout_passes=False),
        out_type=[jax.ShapeDtypeStruct((B * NPG * NPG,), F32),
                  jax.ShapeDtypeStruct((B * NPG * NPG,), F32)],
        scratch_types=[pltpu.VMEM((EH,), jnp.int32),
                       pltpu.VMEM((EH,), jnp.int32),
                       pltpu.VMEM((EH,), F32),
                       pltpu.VMEM((RB * NPG,), F32),
                       pltpu.VMEM((RB * NPG,), F32),
                       pltpu.SemaphoreType.DMA,
                       pltpu.SemaphoreType.DMA,
                       pltpu.SemaphoreType.DMA],
    )
    mw, m1 = f(src, dst, w)
    return (mw.reshape(B, NPG, NPG), m1.reshape(B, NPG, NPG))


def _dot(a, b):
    return jax.lax.dot_general(a, b, (((1,), (0,)), ((), ())),
                               preferred_element_type=F32)


def _dot_t(a, b):
    return jax.lax.dot_general(a, b, (((0,), (0,)), ((), ())),
                               preferred_element_type=F32)


def _unpack_adj(w):
    lo = jax.lax.bitcast_convert_type(lax.shift_left(w, 16), F32)
    hi = jax.lax.bitcast_convert_type(w & jnp.int32(-65536), F32)
    return lo, hi


def _fold(sum_ref, sq_ref, g_ref, be_ref, n):
    m = jnp.sum(sum_ref[...], axis=0) / n
    var = jnp.sum(sq_ref[...], axis=0) / n - m * m
    a = g_ref[...] / jnp.sqrt(var + 1e-5)
    c = be_ref[...] - m * a
    return a, c


def _branch(lo, hi, dis, h, w_ref, b_ref):
    v = _dot(h, w_ref[...])
    vw = dis[:, None] * v
    prop = _dot(lo, vw[:NPG // 2]) + _dot(hi, vw[NPG // 2:])
    u = dis[:, None] * (prop + vw) + b_ref[...]
    return u


def _branch_full(m, dis, h, w_ref, b_ref):
    v = _dot(h, w_ref[...])
    vw = dis[:, None] * v
    u = dis[:, None] * (_dot(m, vw) + vw) + b_ref[...]
    return u


def _adj_halves(ref):
    lo, hi = _unpack_adj(ref[0])
    dis = lax.rsqrt(jnp.sum(lo, axis=1) + jnp.sum(hi, axis=1) + 1.0)
    return lo, hi, dis


def _write_stats(u, u_ref, sum_ref, sq_ref, mx_ref, mn_ref):
    f = u.shape[1]
    u_ref[...] = u
    sum_ref[...] = jnp.sum(u, axis=0).reshape(1, 1, f)
    sq_ref[...] = jnp.sum(u * u, axis=0).reshape(1, 1, f)
    mx_ref[...] = jnp.max(u, axis=0).reshape(1, 1, f)
    mn_ref[...] = jnp.min(u, axis=0).reshape(1, 1, f)


def _stage1_kernel(mw_ref, m1_ref, x_ref, wx_ref, bx_ref, ws_ref, bs_ref,
                   ux_ref, sx_ref, qx_ref, mxx_ref, mnx_ref,
                   us_ref, ss_ref, qs_ref, mxs_ref, mns_ref):
    mw = mw_ref[0]
    disw = lax.rsqrt(jnp.sum(mw, axis=1) + 1.0)
    m1 = m1_ref[0]
    dis1 = lax.rsqrt(jnp.sum(m1, axis=1) + 1.0)
    h = x_ref[...]
    ux = _branch_full(mw, disw, h, wx_ref, bx_ref)
    _write_stats(ux, ux_ref, sx_ref, qx_ref, mxx_ref, mnx_ref)
    us = _branch_full(m1, dis1, h, ws_ref, bs_ref)
    _write_stats(us, us_ref, ss_ref, qs_ref, mxs_ref, mns_ref)


def _stage_kernel(mw_ref, m1_ref, hx_ref, sxp_ref, qxp_ref, gxp_ref, bexp_ref,
                  hs_ref, ssp_ref, qsp_ref, gsp_ref, besp_ref,
                  wx_ref, bx_ref, ws_ref, bs_ref,
                  ux_ref, sx_ref, qx_ref, mxx_ref, mnx_ref,
                  us_ref, ss_ref, qs_ref, mxs_ref, mns_ref):
    mw = mw_ref[0]
    disw = lax.rsqrt(jnp.sum(mw, axis=1) + 1.0)
    m1 = m1_ref[0]
    dis1 = lax.rsqrt(jnp.sum(m1, axis=1) + 1.0)
    ax, cx = _fold(sxp_ref, qxp_ref, gxp_ref, bexp_ref, float(N))
    hx = hx_ref[...] * ax + cx
    ux = _branch_full(mw, disw, hx, wx_ref, bx_ref)
    _write_stats(ux, ux_ref, sx_ref, qx_ref, mxx_ref, mnx_ref)
    as_, cs = _fold(ssp_ref, qsp_ref, gsp_ref, besp_ref, float(N))
    hs = hs_ref[...] * as_ + cs
    us = _branch_full(m1, dis1, hs, ws_ref, bs_ref)
    _write_stats(us, us_ref, ss_ref, qs_ref, mxs_ref, mns_ref)


def _pool_kernel(mw_ref, m1_ref, ux3_ref, sx3_ref, qx3_ref, g13_ref, be13_ref,
                 us1_ref, ss1_ref, qs1_ref, gp1_ref, bep1_ref,
                 us2_ref, ss2_ref, qs2_ref, gp2_ref, bep2_ref,
                 us3_ref, ss3_ref, qs3_ref, gp3_ref, bep3_ref,
                 wpf1_ref, wpf2_ref, wpf3_ref, bpf_ref,
                 padj_ref, px_ref, misc_ref):
    a1, c1 = _fold(ss1_ref, qs1_ref, gp1_ref, bep1_ref, float(N))
    a2, c2 = _fold(ss2_ref, qs2_ref, gp2_ref, bep2_ref, float(N))
    a3, c3 = _fold(ss3_ref, qs3_ref, gp3_ref, bep3_ref, float(N))
    s1 = (_dot(us1_ref[...] * a1 + c1, wpf1_ref[...])
          + _dot(us2_ref[...] * a2 + c2, wpf2_ref[...])
          + _dot(us3_ref[...] * a3 + c3, wpf3_ref[...])
          + bpf_ref[...])
    mx = jnp.max(s1, axis=1, keepdims=True)
    ex = jnp.exp(s1 - mx)
    ss = ex / jnp.sum(ex, axis=1, keepdims=True)
    el = -jnp.sum(ss * jnp.log(ss + 1e-15))
    t1 = _dot(m1_ref[0], ss)
    ml = jnp.sum(ss * t1)
    tw = _dot(mw_ref[0], ss)
    padj_ref[0] = _dot_t(tw, ss)
    ax3, cx3 = _fold(sx3_ref, qx3_ref, g13_ref, be13_ref, float(N))
    x13bn = ux3_ref[...] * ax3 + cx3
    px_ref[0] = _dot_t(ss, x13bn)
    misc_ref[...] = jnp.concatenate(
        [el.reshape(1, 1), ml.reshape(1, 1)], axis=1).reshape(1, 1, 2)


def _maxmin_chunk(mx, mn, a, c):
    return jnp.where(a > 0, a * mx, a * mn) + c


def _head_kernel(padj_ref, px_ref, misc_ref,
                 mxx1_ref, mnx1_ref, sx1_ref, qx1_ref, g11_ref, be11_ref,
                 mxx2_ref, mnx2_ref, sx2_ref, qx2_ref, g12_ref, be12_ref,
                 mxx3_ref, mnx3_ref, sx3_ref, qx3_ref, g13_ref, be13_ref,
                 w21_ref, b21_ref, g21_ref, be21_ref,
                 w22_ref, b22_ref, g22_ref, be22_ref,
                 w23_ref, b23_ref, g23_ref, be23_ref,
                 wf1_ref, bf1_ref, wf2_ref, bf2_ref,
                 out_ref, reg_ref):
    n2 = float(B * C)
    chunks = []
    for mxr, mnr, sr, qr, gr, ber in (
            (mxx1_ref, mnx1_ref, sx1_ref, qx1_ref, g11_ref, be11_ref),
            (mxx2_ref, mnx2_ref, sx2_ref, qx2_ref, g12_ref, be12_ref),
            (mxx3_ref, mnx3_ref, sx3_ref, qx3_ref, g13_ref, be13_ref)):
        a, c = _fold(sr, qr, gr, ber, float(N))
        chunks.append(_maxmin_chunk(mxr[...].reshape(B, -1),
                                    mnr[...].reshape(B, -1), a, c))
    x1_out = jnp.concatenate(chunks, axis=1)

    dis2 = []
    for g in range(B):
        deg = jnp.sum(padj_ref[g], axis=0, keepdims=True) + 1.0
        dis2.append(jnp.where(deg > 0, lax.rsqrt(deg), 0.0))

    def layer2(hs, w_ref, b_ref):
        us = []
        for g in range(B):
            v = _dot(hs[g], w_ref[...])
            vw = dis2[g].reshape(C, 1) * v
            u = dis2[g].reshape(C, 1) * (_dot_t(padj_ref[g], vw) + vw) \
                + b_ref[...]
            us.append(u)
        flat = jnp.concatenate(us, axis=0)
        s = jnp.sum(flat, axis=0, keepdims=True) / n2
        var = jnp.sum(flat * flat, axis=0, keepdims=True) / n2 - s * s
        return us, s, var

    hs = [px_ref[g] for g in range(B)]
    x2_chunks = []
    for w_ref, b_ref, g_ref, be_ref in (
            (w21_ref, b21_ref, g21_ref, be21_ref),
            (w22_ref, b22_ref, g22_ref, be22_ref),
            (w23_ref, b23_ref, g23_ref, be23_ref)):
        us, m, var = layer2(hs, w_ref, b_ref)
        a = g_ref[...] / jnp.sqrt(var + 1e-5)
        c = be_ref[...] - m * a
        mxs = jnp.concatenate(
            [jnp.max(u, axis=0, keepdims=True) for u in us], axis=0)
        mns = jnp.concatenate(
            [jnp.min(u, axis=0, keepdims=True) for u in us], axis=0)
        x2_chunks.append(_maxmin_chunk(mxs, mns, a, c))
        hs = [u * a + c for u in us]
    x2_out = jnp.concatenate(x2_chunks, axis=1)

    conv = jnp.concatenate([x1_out, x2_out], axis=1)
    h = jnp.maximum(_dot(conv, wf1_ref[...]) + bf1_ref[...], 0.0)
    out_ref[...] = _dot(h, wf2_ref[...]) + bf2_ref[...]
    misc = misc_ref[...].reshape(B, 2)
    reg = (jnp.sum(misc[:, 0]) / float(N)) - (jnp.sum(misc[:, 1]) / float(E))
    reg_ref[...] = reg.reshape(1, 1)


def _full(shape):
    nd = len(shape)
    return pl.BlockSpec(shape, lambda g, _nd=nd: (0,) * _nd)


def _gblk(shape):
    nd = len(shape)
    return pl.BlockSpec((1,) + shape[1:],
                        lambda g, _nd=nd: (g,) + (0,) * (_nd - 1))


def _nblk(f):
    return pl.BlockSpec((NPG, f), lambda g: (g, 0))


def _stage_out(fx, fs):
    shapes = [jax.ShapeDtypeStruct((N, fx), F32)] + \
             [jax.ShapeDtypeStruct((B, 1, fx), F32)] * 4 + \
             [jax.ShapeDtypeStruct((N, fs), F32)] + \
             [jax.ShapeDtypeStruct((B, 1, fs), F32)] * 4
    specs = [_nblk(fx)] + [_gblk((B, 1, fx))] * 4 + \
            [_nblk(fs)] + [_gblk((B, 1, fs))] * 4
    return shapes, specs


def kernel(x, edge_index, edge_attr, params):
    p = params
    mw, m1 = _sc_build(edge_index[0], edge_index[1], edge_attr)

    def row(name):
        return p[name].reshape(1, -1)

    adj_spec = pl.BlockSpec((1, NPG, NPG), lambda g: (g, 0, 0))
    adj1_spec = adj_spec

    shapes, ospecs = _stage_out(30, 30)
    s1out = pl.pallas_call(
        _stage1_kernel,
        grid=(B,),
        in_specs=[adj_spec, adj1_spec, _nblk(3),
                  _full((3, 30)), _full((1, 30)),
                  _full((3, 30)), _full((1, 30))],
        out_specs=ospecs,
        out_shape=shapes,
    )(mw, m1, x, p['W11'], row('b11'), p['Wp11'], row('bp11'))
    (ux1, sx1, qx1, mxx1, mnx1, us1, ss1, qs1, mxs1, mns1) = s1out

    def stage(fx_in, fs_in, fx_out, fs_out, hx, sxp, qxp, gxp, bexp,
              hs, ssp, qsp, gsp, besp, wx, bx, ws, bs):
        shapes, ospecs = _stage_out(fx_out, fs_out)
        return pl.pallas_call(
            _stage_kernel,
            grid=(B,),
            in_specs=[adj_spec, adj1_spec,
                      _nblk(fx_in), _full((B, 1, fx_in)), _full((B, 1, fx_in)),
                      _full((1, fx_in)), _full((1, fx_in)),
                      _nblk(fs_in), _full((B, 1, fs_in)), _full((B, 1, fs_in)),
                      _full((1, fs_in)), _full((1, fs_in)),
                      _full((fx_in, fx_out)), _full((1, fx_out)),
                      _full((fs_in, fs_out)), _full((1, fs_out))],
            out_specs=ospecs,
            out_shape=shapes,
        )(mw, m1, hx, sxp, qxp, gxp, bexp, hs, ssp, qsp, gsp, besp,
          wx, bx, ws, bs)

    (ux2, sx2, qx2, mxx2, mnx2, us2, ss2, qs2, _, _) = stage(
        30, 30, 30, 30, ux1, sx1, qx1, row('g11'), row('be11'),
        us1, ss1, qs1, row('gp11'), row('bep11'),
        p['W12'], row('b12'), p['Wp12'], row('bp12'))
    (ux3, sx3, qx3, mxx3, mnx3, us3, ss3, qs3, _, _) = stage(
        30, 30, 30, 100, ux2, sx2, qx2, row('g12'), row('be12'),
        us2, ss2, qs2, row('gp12'), row('bep12'),
        p['W13'], row('b13'), p['Wp13'], row('bp13'))

    wpf = p['Wpf']
    padj, px, misc = pl.pallas_call(
        _pool_kernel,
        grid=(B,),
        in_specs=[adj_spec, adj1_spec,
                  _nblk(30), _full((B, 1, 30)), _full((B, 1, 30)),
                  _full((1, 30)), _full((1, 30)),
                  _nblk(30), _full((B, 1, 30)), _full((B, 1, 30)),
                  _full((1, 30)), _full((1, 30)),
                  _nblk(30), _full((B, 1, 30)), _full((B, 1, 30)),
                  _full((1, 30)), _full((1, 30)),
                  _nblk(100), _full((B, 1, 100)), _full((B, 1, 100)),
                  _full((1, 100)), _full((1, 100)),
                  _full((30, 100)), _full((30, 100)), _full((100, 100)),
                  _full((1, 100))],
        out_specs=[_gblk((B, C, C)), _gblk((B, C, 30)), _gblk((B, 1, 2))],
        out_shape=[jax.ShapeDtypeStruct((B, C, C), F32),
                   jax.ShapeDtypeStruct((B, C, 30), F32),
                   jax.ShapeDtypeStruct((B, 1, 2), F32)],
    )(mw, m1, ux3, sx3, qx3, row('g13'), row('be13'),
      us1, ss1, qs1, row('gp11'), row('bep11'),
      us2, ss2, qs2, row('gp12'), row('bep12'),
      us3, ss3, qs3, row('gp13'), row('bep13'),
      wpf[0:30], wpf[30:60], wpf[60:160], row('bpf'))

    args = [padj, px, misc,
            mxx1, mnx1, sx1, qx1, row('g11'), row('be11'),
            mxx2, mnx2, sx2, qx2, row('g12'), row('be12'),
            mxx3, mnx3, sx3, qx3, row('g13'), row('be13'),
            p['W21'], row('b21'), row('g21'), row('be21'),
            p['W22'], row('b22'), row('g22'), row('be22'),
            p['W23'], row('b23'), row('g23'), row('be23'),
            p['Wf1'], row('bf1'), p['Wf2'], row('bf2')]
    out, reg = pl.pallas_call(
        _head_kernel,
        out_shape=[jax.ShapeDtypeStruct((B, 6), F32),
                   jax.ShapeDtypeStruct((1, 1), F32)],
    )(*args)
    return (out, reg.reshape(()))
